# baseline (speedup 1.0000x reference)
import functools

import jax
import jax.numpy as jnp
from jax import lax
from jax.experimental import pallas as pl
from jax.experimental.pallas import tpu as pltpu
from jax.experimental.pallas import tpu_sc as plsc

N_NODES = 10000
N_EDGES = 320000
D = 128
HID = 128
OUT = 10
GRID_SIZE = 5
SPLINE_ORDER = 3
COEF = GRID_SIZE + SPLINE_ORDER

NC = 2
NS = 16
NW = NC * NS
CHUNK = 128
CNT0 = 160
GRP = 8
E_PAD = NS * CNT0 * CHUNK
ACC_ROWS = 10112
ROWS_PER_TILE = ACC_ROWS // NS

KNOTS = [0.4 * i - 2.2 for i in range(GRID_SIZE + 2 * SPLINE_ORDER + 1)]


NB = 2


STRIPE_PIECES = [CHUNK] * (ROWS_PER_TILE // CHUNK) + [ROWS_PER_TILE % CHUNK]


def _sc_scatter_body(h_hbm, row_hbm, col_hbm, out_hbm,
                     row_v, col_v, bufs, gsems, isems, jsems, acc_sh):
  c = lax.axis_index("c")
  s = lax.axis_index("s")
  wid = s

  @pl.when(c == 0)
  def _sc0_work():
    _sc_scatter_tile(h_hbm, row_hbm, col_hbm, out_hbm,
                     row_v, col_v, bufs, gsems, isems, jsems, acc_sh, s, wid)


def _sc_scatter_tile(h_hbm, row_hbm, col_hbm, out_hbm,
                     row_v, col_v, bufs, gsems, isems, jsems, acc_sh, s, wid):
  def zrow(r, carry):
    for q in range(D // 16):
      bufs[0][r, pl.ds(q * 16, 16)] = jnp.zeros((16,), jnp.float32)
    return carry

  lax.fori_loop(0, CHUNK, zrow, 0, unroll=False)
  off = s * ROWS_PER_TILE
  for n in STRIPE_PIECES:
    pltpu.sync_copy(bufs[0].at[pl.ds(0, n)], acc_sh.at[pl.ds(off, n)])
    off = off + n
  plsc.subcore_barrier()

  def idx_start(sl, g):
    pltpu.async_copy(row_hbm.at[wid, pl.ds(g * GRP, GRP)], row_v.at[sl],
                     isems[sl])
    pltpu.async_copy(col_hbm.at[wid, pl.ds(g * GRP, GRP)], col_v.at[sl],
                     jsems[sl])

  def idx_wait(sl, g):
    pltpu.make_async_copy(row_hbm.at[wid, pl.ds(g * GRP, GRP)],
                          row_v.at[sl], isems[sl]).wait()
    pltpu.make_async_copy(col_hbm.at[wid, pl.ds(g * GRP, GRP)],
                          col_v.at[sl], jsems[sl]).wait()

  idx_start(0, 0)

  def pair(p, carry):
    for sl in range(2):
      g = 2 * p + sl
      idx_wait(sl, g)
      idx_start(1 - sl, g + 1)
      descs = [None] * GRP
      for b in range(NB):
        descs[b] = pltpu.async_copy(h_hbm.at[col_v.at[sl, b]], bufs[b],
                                    gsems[b])
      for b in range(GRP):
        bsl = b % NB
        descs[b].wait()
        pltpu.sync_copy(bufs[bsl], acc_sh.at[row_v.at[sl, b]], add=True)
        if b + NB < GRP:
          descs[b + NB] = pltpu.async_copy(h_hbm.at[col_v.at[sl, b + NB]],
                                           bufs[bsl], gsems[bsl])
    return carry

  n_pairs = CNT0 // (2 * GRP)
  lax.fori_loop(0, n_pairs, pair, 0, unroll=False)
  idx_wait(0, 2 * n_pairs)
  plsc.subcore_barrier()

  off = s * ROWS_PER_TILE
  wdescs = [None, None]
  for q, n in enumerate(STRIPE_PIECES):
    bsl = q % NB
    if wdescs[bsl] is not None:
      wdescs[bsl].wait()
    pltpu.sync_copy(acc_sh.at[pl.ds(off, n)], bufs[bsl].at[pl.ds(0, n)])
    wdescs[bsl] = pltpu.async_copy(bufs[bsl].at[pl.ds(0, n)],
                                   out_hbm.at[pl.ds(off, n)], gsems[bsl])
    off = off + n
  for d in wdescs:
    if d is not None:
      d.wait()


@jax.jit
def _sc_scatter(h, row_r, col_r):
  mesh = plsc.VectorSubcoreMesh(core_axis_name="c", subcore_axis_name="s")
  fn = pl.kernel(
      _sc_scatter_body,
      out_type=jax.ShapeDtypeStruct((ACC_ROWS, D), jnp.float32),
      mesh=mesh,
      scratch_types=[
          pltpu.VMEM((2, GRP, CHUNK), jnp.int32),
          pltpu.VMEM((2, GRP, CHUNK), jnp.int32),
          tuple(pltpu.VMEM((CHUNK, D), jnp.float32) for _ in range(NB)),
          tuple(pltpu.SemaphoreType.DMA for _ in range(NB)),
          tuple(pltpu.SemaphoreType.DMA for _ in range(2)),
          tuple(pltpu.SemaphoreType.DMA for _ in range(2)),
          pltpu.VMEM_SHARED((ACC_ROWS, D), jnp.float32),
      ],
  )
  return fn(h, row_r, col_r)


def _bspline_bases(x):
  g = KNOTS
  b = [jnp.where((x >= g[j]) & (x < g[j + 1]), 1.0, 0.0).astype(x.dtype)
       for j in range(len(g) - 1)]
  for k in range(1, SPLINE_ORDER + 1):
    nb = []
    for j in range(len(b) - 1):
      left = (x - g[j]) * (1.0 / (g[j + k] - g[j])) * b[j]
      right = (g[j + k + 1] - x) * (1.0 / (g[j + k + 1] - g[j + 1])) * b[j + 1]
      nb.append(left + right)
    b = nb
  return b


def _dense1_body(scale_ref, p0_ref, h_ref, bwt_ref, swt_ref,
                 y_ref, sums_ref):
  pid = pl.program_id(0)
  pooled = p0_ref[...] + scale_ref[0] * h_ref[...]
  silu = pooled * jax.nn.sigmoid(pooled)
  y = jnp.dot(silu, bwt_ref[...], preferred_element_type=jnp.float32)
  bases = _bspline_bases(pooled)
  for j in range(COEF):
    y = y + jnp.dot(bases[j], swt_ref[j], preferred_element_type=jnp.float32)
  y_ref[...] = y

  ssum = jnp.sum(y, axis=0)
  ssq = jnp.sum(y * y, axis=0)
  upd = jnp.concatenate(
      [ssum[None], ssq[None], jnp.zeros((6, y.shape[1]), jnp.float32)], axis=0)

  @pl.when(pid == 0)
  def _():
    sums_ref[...] = jnp.zeros_like(sums_ref)

  sums_ref[...] += upd


def _dense1(scale, p0, h, bwt, swt, block):
  nblk = N_NODES // block
  return pl.pallas_call(
      _dense1_body,
      grid=(nblk,),
      in_specs=[
          pl.BlockSpec(memory_space=pltpu.SMEM),
          pl.BlockSpec((block, D), lambda i: (i, 0)),
          pl.BlockSpec((block, D), lambda i: (i, 0)),
          pl.BlockSpec((D, HID), lambda i: (0, 0)),
          pl.BlockSpec((COEF, D, HID), lambda i: (0, 0, 0)),
      ],
      out_specs=[
          pl.BlockSpec((block, HID), lambda i: (i, 0)),
          pl.BlockSpec((8, HID), lambda i: (0, 0)),
      ],
      out_shape=[
          jax.ShapeDtypeStruct((N_NODES, HID), jnp.float32),
          jax.ShapeDtypeStruct((8, HID), jnp.float32),
      ],
  )(scale, p0, h, bwt, swt)


def _bn_relu_body(y_ref, sums_ref, gamma_ref, beta_ref, o_ref):
  inv_n = 1.0 / N_NODES
  mean = sums_ref[0:1, :] * inv_n
  var = sums_ref[1:2, :] * inv_n - mean * mean
  inv = lax.rsqrt(var + 1e-5) * gamma_ref[...]
  o_ref[...] = jnp.maximum((y_ref[...] - mean) * inv + beta_ref[...], 0.0)


def _bn_relu(y, sums, gamma, beta, block):
  nblk = N_NODES // block
  return pl.pallas_call(
      _bn_relu_body,
      grid=(nblk,),
      in_specs=[
          pl.BlockSpec((block, HID), lambda i: (i, 0)),
          pl.BlockSpec((8, HID), lambda i: (0, 0)),
          pl.BlockSpec((1, HID), lambda i: (0, 0)),
          pl.BlockSpec((1, HID), lambda i: (0, 0)),
      ],
      out_specs=pl.BlockSpec((block, HID), lambda i: (i, 0)),
      out_shape=jax.ShapeDtypeStruct((N_NODES, HID), jnp.float32),
  )(y, sums, gamma, beta)


def _bn_relu_clf_body(y_ref, sums_ref, gamma_ref, beta_ref, cw_ref, cb_ref,
                      o_ref):
  inv_n = 1.0 / N_NODES
  mean = sums_ref[0:1, :] * inv_n
  var = sums_ref[1:2, :] * inv_n - mean * mean
  inv = lax.rsqrt(var + 1e-5) * gamma_ref[...]
  h = jnp.maximum((y_ref[...] - mean) * inv + beta_ref[...], 0.0)
  o_ref[...] = jnp.dot(h, cw_ref[...], preferred_element_type=jnp.float32) \
      + cb_ref[...]


def _bn_relu_clf(y, sums, gamma, beta, cw_pad, cb_pad, block):
  nblk = N_NODES // block
  return pl.pallas_call(
      _bn_relu_clf_body,
      grid=(nblk,),
      in_specs=[
          pl.BlockSpec((block, HID), lambda i: (i, 0)),
          pl.BlockSpec((8, HID), lambda i: (0, 0)),
          pl.BlockSpec((1, HID), lambda i: (0, 0)),
          pl.BlockSpec((1, HID), lambda i: (0, 0)),
          pl.BlockSpec((HID, 128), lambda i: (0, 0)),
          pl.BlockSpec((1, 128), lambda i: (0, 0)),
      ],
      out_specs=pl.BlockSpec((block, 128), lambda i: (i, 0)),
      out_shape=jax.ShapeDtypeStruct((N_NODES, 128), jnp.float32),
  )(y, sums, gamma, beta, cw_pad, cb_pad)


BLOCK = 1000


def kernel(x, edge_index, eps, kan0_base, kan0_spline, kan1_base, kan1_spline,
           bn0_gamma, bn0_beta, bn1_gamma, bn1_beta, clf_w, clf_b):
  row = edge_index[0].astype(jnp.int32)
  col = edge_index[1].astype(jnp.int32)
  pad = E_PAD - N_EDGES
  dump_rows = N_NODES + jnp.arange(pad, dtype=jnp.int32) % (ACC_ROWS - N_NODES)

  def _split(v, tail):
    out = jnp.concatenate([v, tail]).reshape(NS, CNT0, CHUNK)
    return jnp.pad(out, ((0, 0), (0, GRP), (0, 0)))

  row_r = _split(row, dump_rows)
  col_r = _split(col, jnp.zeros((pad,), jnp.int32))

  bwts = [kan0_base.T, kan1_base.T]
  swts = [kan0_spline.transpose(2, 1, 0), kan1_spline.transpose(2, 1, 0)]
  gammas = [bn0_gamma.reshape(1, HID), bn1_gamma.reshape(1, HID)]
  betas = [bn0_beta.reshape(1, HID), bn1_beta.reshape(1, HID)]
  cw_pad = jnp.zeros((HID, 128), jnp.float32).at[:, :OUT].set(clf_w.T)
  cb_pad = jnp.zeros((1, 128), jnp.float32).at[0, :OUT].set(clf_b)

  h = x
  for layer in range(2):
    pooled0 = _sc_scatter(h, row_r, col_r)[:N_NODES]
    scale = (1.0 + eps[layer]).reshape(1)
    y, sums = _dense1(scale, pooled0, h, bwts[layer], swts[layer], BLOCK)
    if layer == 0:
      h = _bn_relu(y, sums, gammas[layer], betas[layer], BLOCK)
    else:
      logits_pad = _bn_relu_clf(y, sums, gammas[layer], betas[layer],
                                cw_pad, cb_pad, BLOCK)
  return logits_pad[:, :OUT]


# speedup vs baseline: 1.1071x; 1.1071x over previous
import functools

import jax
import jax.numpy as jnp
from jax import lax
from jax.experimental import pallas as pl
from jax.experimental.pallas import tpu as pltpu
from jax.experimental.pallas import tpu_sc as plsc

N_NODES = 10000
N_EDGES = 320000
D = 128
HID = 128
OUT = 10
GRID_SIZE = 5
SPLINE_ORDER = 3
COEF = GRID_SIZE + SPLINE_ORDER

NC = 2
NS = 16
NW = NC * NS
CHUNK = 128
CNT = 80
GRP = 8
E_PAD = NW * CNT * CHUNK
H_AUG = N_NODES + 8
ACC_ROWS = 10112
ROWS_PER_TILE = ACC_ROWS // NS

KNOTS = [0.4 * i - 2.2 for i in range(GRID_SIZE + 2 * SPLINE_ORDER + 1)]


NB = 2


STRIPE_PIECES = [CHUNK] * (ROWS_PER_TILE // CHUNK) + [ROWS_PER_TILE % CHUNK]


def _sc_scatter_body(h_hbm, row_hbm, col_hbm, out_hbm,
                     row_v, col_v, bufs, gsems, isems, jsems, acc_sh):
  c = lax.axis_index("c")
  s = lax.axis_index("s")
  wid = c * NS + s

  def zrow(r, carry):
    for q in range(D // 16):
      bufs[0][r, pl.ds(q * 16, 16)] = jnp.zeros((16,), jnp.float32)
    return carry

  lax.fori_loop(0, CHUNK, zrow, 0, unroll=False)
  off = s * ROWS_PER_TILE
  for n in STRIPE_PIECES:
    pltpu.sync_copy(bufs[0].at[pl.ds(0, n)], acc_sh.at[pl.ds(off, n)])
    off = off + n
  plsc.subcore_barrier()

  def idx_start(sl, g):
    pltpu.async_copy(row_hbm.at[wid, pl.ds(g * GRP, GRP)], row_v.at[sl],
                     isems[sl])
    pltpu.async_copy(col_hbm.at[wid, pl.ds(g * GRP, GRP)], col_v.at[sl],
                     jsems[sl])

  def idx_wait(sl, g):
    pltpu.make_async_copy(row_hbm.at[wid, pl.ds(g * GRP, GRP)],
                          row_v.at[sl], isems[sl]).wait()
    pltpu.make_async_copy(col_hbm.at[wid, pl.ds(g * GRP, GRP)],
                          col_v.at[sl], jsems[sl]).wait()

  idx_start(0, 0)

  def pair(p, carry):
    for sl in range(2):
      g = 2 * p + sl
      idx_wait(sl, g)
      idx_start(1 - sl, g + 1)
      descs = [None] * GRP
      for b in range(NB):
        descs[b] = pltpu.async_copy(h_hbm.at[col_v.at[sl, b]], bufs[b],
                                    gsems[b])
      for b in range(GRP):
        bsl = b % NB
        descs[b].wait()
        pltpu.sync_copy(bufs[bsl], acc_sh.at[row_v.at[sl, b]], add=True)
        if b + NB < GRP:
          descs[b + NB] = pltpu.async_copy(h_hbm.at[col_v.at[sl, b + NB]],
                                           bufs[bsl], gsems[bsl])
    return carry

  n_pairs = CNT // (2 * GRP)
  lax.fori_loop(0, n_pairs, pair, 0, unroll=False)
  idx_wait(0, 2 * n_pairs)
  plsc.subcore_barrier()

  off = s * ROWS_PER_TILE
  wdescs = [None, None]
  for q, n in enumerate(STRIPE_PIECES):
    bsl = q % NB
    if wdescs[bsl] is not None:
      wdescs[bsl].wait()
    pltpu.sync_copy(acc_sh.at[pl.ds(off, n)], bufs[bsl].at[pl.ds(0, n)])
    wdescs[bsl] = pltpu.async_copy(bufs[bsl].at[pl.ds(0, n)],
                                   out_hbm.at[c, pl.ds(off, n)], gsems[bsl])
    off = off + n
  for d in wdescs:
    if d is not None:
      d.wait()


@jax.jit
def _sc_scatter(h, row_r, col_r):
  mesh = plsc.VectorSubcoreMesh(core_axis_name="c", subcore_axis_name="s")
  fn = pl.kernel(
      _sc_scatter_body,
      out_type=jax.ShapeDtypeStruct((NC, ACC_ROWS, D), jnp.float32),
      mesh=mesh,
      scratch_types=[
          pltpu.VMEM((2, GRP, CHUNK), jnp.int32),
          pltpu.VMEM((2, GRP, CHUNK), jnp.int32),
          tuple(pltpu.VMEM((CHUNK, D), jnp.float32) for _ in range(NB)),
          tuple(pltpu.SemaphoreType.DMA for _ in range(NB)),
          tuple(pltpu.SemaphoreType.DMA for _ in range(2)),
          tuple(pltpu.SemaphoreType.DMA for _ in range(2)),
          pltpu.VMEM_SHARED((ACC_ROWS, D), jnp.float32),
      ],
  )
  return fn(h, row_r, col_r)


def _bspline_bases(x):
  g = KNOTS
  b = [jnp.where((x >= g[j]) & (x < g[j + 1]), 1.0, 0.0).astype(x.dtype)
       for j in range(len(g) - 1)]
  for k in range(1, SPLINE_ORDER + 1):
    nb = []
    for j in range(len(b) - 1):
      left = (x - g[j]) * (1.0 / (g[j + k] - g[j])) * b[j]
      right = (g[j + k + 1] - x) * (1.0 / (g[j + k + 1] - g[j + 1])) * b[j + 1]
      nb.append(left + right)
    b = nb
  return b


def _dense1_body(scale_ref, p0_ref, p1_ref, h_ref, bwt_ref, swt_ref,
                 y_ref, sums_ref):
  pid = pl.program_id(0)
  pooled = p0_ref[...] + p1_ref[...] + scale_ref[0] * h_ref[...]
  silu = pooled * jax.nn.sigmoid(pooled)
  y = jnp.dot(silu, bwt_ref[...], preferred_element_type=jnp.float32)
  bases = _bspline_bases(pooled)
  for j in range(COEF):
    y = y + jnp.dot(bases[j], swt_ref[j], preferred_element_type=jnp.float32)
  y_ref[...] = y

  ssum = jnp.sum(y, axis=0)
  ssq = jnp.sum(y * y, axis=0)
  upd = jnp.concatenate(
      [ssum[None], ssq[None], jnp.zeros((6, y.shape[1]), jnp.float32)], axis=0)

  @pl.when(pid == 0)
  def _():
    sums_ref[...] = jnp.zeros_like(sums_ref)

  sums_ref[...] += upd


def _dense1(scale, p0, p1, h, bwt, swt, block):
  nblk = N_NODES // block
  return pl.pallas_call(
      _dense1_body,
      grid=(nblk,),
      in_specs=[
          pl.BlockSpec(memory_space=pltpu.SMEM),
          pl.BlockSpec((block, D), lambda i: (i, 0)),
          pl.BlockSpec((block, D), lambda i: (i, 0)),
          pl.BlockSpec((block, D), lambda i: (i, 0)),
          pl.BlockSpec((D, HID), lambda i: (0, 0)),
          pl.BlockSpec((COEF, D, HID), lambda i: (0, 0, 0)),
      ],
      out_specs=[
          pl.BlockSpec((block, HID), lambda i: (i, 0)),
          pl.BlockSpec((8, HID), lambda i: (0, 0)),
      ],
      out_shape=[
          jax.ShapeDtypeStruct((N_NODES, HID), jnp.float32),
          jax.ShapeDtypeStruct((8, HID), jnp.float32),
      ],
  )(scale, p0, p1, h, bwt, swt)


def _bn_relu_body(y_ref, sums_ref, gamma_ref, beta_ref, o_ref):
  inv_n = 1.0 / N_NODES
  mean = sums_ref[0:1, :] * inv_n
  var = sums_ref[1:2, :] * inv_n - mean * mean
  inv = lax.rsqrt(var + 1e-5) * gamma_ref[...]
  o_ref[...] = jnp.maximum((y_ref[...] - mean) * inv + beta_ref[...], 0.0)


def _bn_relu(y, sums, gamma, beta, block):
  nblk = N_NODES // block
  return pl.pallas_call(
      _bn_relu_body,
      grid=(nblk,),
      in_specs=[
          pl.BlockSpec((block, HID), lambda i: (i, 0)),
          pl.BlockSpec((8, HID), lambda i: (0, 0)),
          pl.BlockSpec((1, HID), lambda i: (0, 0)),
          pl.BlockSpec((1, HID), lambda i: (0, 0)),
      ],
      out_specs=pl.BlockSpec((block, HID), lambda i: (i, 0)),
      out_shape=jax.ShapeDtypeStruct((N_NODES, HID), jnp.float32),
  )(y, sums, gamma, beta)


def _bn_relu_clf_body(y_ref, sums_ref, gamma_ref, beta_ref, cw_ref, cb_ref,
                      o_ref):
  inv_n = 1.0 / N_NODES
  mean = sums_ref[0:1, :] * inv_n
  var = sums_ref[1:2, :] * inv_n - mean * mean
  inv = lax.rsqrt(var + 1e-5) * gamma_ref[...]
  h = jnp.maximum((y_ref[...] - mean) * inv + beta_ref[...], 0.0)
  o_ref[...] = jnp.dot(h, cw_ref[...], preferred_element_type=jnp.float32) \
      + cb_ref[...]


def _bn_relu_clf(y, sums, gamma, beta, cw_pad, cb_pad, block):
  nblk = N_NODES // block
  return pl.pallas_call(
      _bn_relu_clf_body,
      grid=(nblk,),
      in_specs=[
          pl.BlockSpec((block, HID), lambda i: (i, 0)),
          pl.BlockSpec((8, HID), lambda i: (0, 0)),
          pl.BlockSpec((1, HID), lambda i: (0, 0)),
          pl.BlockSpec((1, HID), lambda i: (0, 0)),
          pl.BlockSpec((HID, 128), lambda i: (0, 0)),
          pl.BlockSpec((1, 128), lambda i: (0, 0)),
      ],
      out_specs=pl.BlockSpec((block, 128), lambda i: (i, 0)),
      out_shape=jax.ShapeDtypeStruct((N_NODES, 128), jnp.float32),
  )(y, sums, gamma, beta, cw_pad, cb_pad)


BLOCK = 1000


def kernel(x, edge_index, eps, kan0_base, kan0_spline, kan1_base, kan1_spline,
           bn0_gamma, bn0_beta, bn1_gamma, bn1_beta, clf_w, clf_b):
  row = edge_index[0].astype(jnp.int32)
  col = edge_index[1].astype(jnp.int32)
  pad = E_PAD - N_EDGES
  pad_rows = jnp.arange(pad, dtype=jnp.int32) % N_NODES

  def _split(v, tail):
    out = jnp.concatenate([v, tail]).reshape(NW, CNT, CHUNK)
    return jnp.pad(out, ((0, 0), (0, GRP), (0, 0)))

  row_r = _split(row, pad_rows)
  col_r = _split(col, jnp.full((pad,), N_NODES, jnp.int32))
  zrows = jnp.zeros((H_AUG - N_NODES, D), jnp.float32)

  bwts = [kan0_base.T, kan1_base.T]
  swts = [kan0_spline.transpose(2, 1, 0), kan1_spline.transpose(2, 1, 0)]
  gammas = [bn0_gamma.reshape(1, HID), bn1_gamma.reshape(1, HID)]
  betas = [bn0_beta.reshape(1, HID), bn1_beta.reshape(1, HID)]
  cw_pad = jnp.zeros((HID, 128), jnp.float32).at[:, :OUT].set(clf_w.T)
  cb_pad = jnp.zeros((1, 128), jnp.float32).at[0, :OUT].set(clf_b)

  h = x
  for layer in range(2):
    h_aug = jnp.concatenate([h, zrows], axis=0)
    partials = _sc_scatter(h_aug, row_r, col_r)[:, :N_NODES]
    scale = (1.0 + eps[layer]).reshape(1)
    y, sums = _dense1(scale, partials[0], partials[1], h,
                      bwts[layer], swts[layer], BLOCK)
    if layer == 0:
      h = _bn_relu(y, sums, gammas[layer], betas[layer], BLOCK)
    else:
      logits_pad = _bn_relu_clf(y, sums, gammas[layer], betas[layer],
                                cw_pad, cb_pad, BLOCK)
  return logits_pad[:, :OUT]


# speedup vs baseline: 3.2504x; 2.9358x over previous
import functools

import jax
import jax.numpy as jnp
from jax import lax
from jax.experimental import pallas as pl
from jax.experimental.pallas import tpu as pltpu
from jax.experimental.pallas import tpu_sc as plsc

N_NODES = 10000
N_EDGES = 320000
D = 128
HID = 128
OUT = 10
GRID_SIZE = 5
SPLINE_ORDER = 3
COEF = GRID_SIZE + SPLINE_ORDER

NC = 2
NS = 16
NW = NC * NS
CHUNK = 128
CNT = 80
GRP = 8
E_PAD = NW * CNT * CHUNK
H_AUG = N_NODES + (E_PAD - N_EDGES)
ACC_ROWS = 10112
ROWS_PER_TILE = ACC_ROWS // NS

KNOTS = [0.4 * i - 2.2 for i in range(GRID_SIZE + 2 * SPLINE_ORDER + 1)]


NB = 2


STRIPE_PIECES = [CHUNK] * (ROWS_PER_TILE // CHUNK) + [ROWS_PER_TILE % CHUNK]


def _sc_scatter_body(h_hbm, row_hbm, col_hbm, out_hbm,
                     row_v, col_v, bufs, gsems, isems, jsems, acc_sh):
  c = lax.axis_index("c")
  s = lax.axis_index("s")
  wid = c * NS + s

  def zrow(r, carry):
    for q in range(D // 16):
      bufs[0][r, pl.ds(q * 16, 16)] = jnp.zeros((16,), jnp.float32)
    return carry

  lax.fori_loop(0, CHUNK, zrow, 0, unroll=False)
  off = s * ROWS_PER_TILE
  for n in STRIPE_PIECES:
    pltpu.sync_copy(bufs[0].at[pl.ds(0, n)], acc_sh.at[pl.ds(off, n)])
    off = off + n
  plsc.subcore_barrier()

  def idx_start(sl, g):
    pltpu.async_copy(row_hbm.at[wid, pl.ds(g * GRP, GRP)], row_v.at[sl],
                     isems[sl])
    pltpu.async_copy(col_hbm.at[wid, pl.ds(g * GRP, GRP)], col_v.at[sl],
                     jsems[sl])

  def idx_wait(sl, g):
    pltpu.make_async_copy(row_hbm.at[wid, pl.ds(g * GRP, GRP)],
                          row_v.at[sl], isems[sl]).wait()
    pltpu.make_async_copy(col_hbm.at[wid, pl.ds(g * GRP, GRP)],
                          col_v.at[sl], jsems[sl]).wait()

  idx_start(0, 0)

  def pair(p, carry):
    for sl in range(2):
      g = 2 * p + sl
      idx_wait(sl, g)
      idx_start(1 - sl, g + 1)
      descs = [None] * GRP
      for b in range(NB):
        descs[b] = pltpu.async_copy(h_hbm.at[col_v.at[sl, b]], bufs[b],
                                    gsems[b])
      for b in range(GRP):
        bsl = b % NB
        descs[b].wait()
        pltpu.sync_copy(bufs[bsl], acc_sh.at[row_v.at[sl, b]], add=True)
        if b + NB < GRP:
          descs[b + NB] = pltpu.async_copy(h_hbm.at[col_v.at[sl, b + NB]],
                                           bufs[bsl], gsems[bsl])
    return carry

  n_pairs = CNT // (2 * GRP)
  lax.fori_loop(0, n_pairs, pair, 0, unroll=False)
  idx_wait(0, 2 * n_pairs)
  plsc.subcore_barrier()

  off = s * ROWS_PER_TILE
  wdescs = [None, None]
  for q, n in enumerate(STRIPE_PIECES):
    bsl = q % NB
    if wdescs[bsl] is not None:
      wdescs[bsl].wait()
    pltpu.sync_copy(acc_sh.at[pl.ds(off, n)], bufs[bsl].at[pl.ds(0, n)])
    wdescs[bsl] = pltpu.async_copy(bufs[bsl].at[pl.ds(0, n)],
                                   out_hbm.at[c, pl.ds(off, n)], gsems[bsl])
    off = off + n
  for d in wdescs:
    if d is not None:
      d.wait()


@jax.jit
def _sc_scatter(h, row_r, col_r):
  mesh = plsc.VectorSubcoreMesh(core_axis_name="c", subcore_axis_name="s")
  fn = pl.kernel(
      _sc_scatter_body,
      out_type=jax.ShapeDtypeStruct((NC, ACC_ROWS, D), jnp.float32),
      mesh=mesh,
      scratch_types=[
          pltpu.VMEM((2, GRP, CHUNK), jnp.int32),
          pltpu.VMEM((2, GRP, CHUNK), jnp.int32),
          tuple(pltpu.VMEM((CHUNK, D), jnp.float32) for _ in range(NB)),
          tuple(pltpu.SemaphoreType.DMA for _ in range(NB)),
          tuple(pltpu.SemaphoreType.DMA for _ in range(2)),
          tuple(pltpu.SemaphoreType.DMA for _ in range(2)),
          pltpu.VMEM_SHARED((ACC_ROWS, D), jnp.float32),
      ],
  )
  return fn(h, row_r, col_r)


def _bspline_bases(x):
  g = KNOTS
  b = [jnp.where((x >= g[j]) & (x < g[j + 1]), 1.0, 0.0).astype(x.dtype)
       for j in range(len(g) - 1)]
  for k in range(1, SPLINE_ORDER + 1):
    nb = []
    for j in range(len(b) - 1):
      left = (x - g[j]) * (1.0 / (g[j + k] - g[j])) * b[j]
      right = (g[j + k + 1] - x) * (1.0 / (g[j + k + 1] - g[j + 1])) * b[j + 1]
      nb.append(left + right)
    b = nb
  return b


def _dense1_body(scale_ref, p0_ref, p1_ref, h_ref, bwt_ref, swt_ref,
                 y_ref, sums_ref):
  pid = pl.program_id(0)
  pooled = p0_ref[...] + p1_ref[...] + scale_ref[0] * h_ref[...]
  silu = pooled * jax.nn.sigmoid(pooled)
  y = jnp.dot(silu, bwt_ref[...], preferred_element_type=jnp.float32)
  bases = _bspline_bases(pooled)
  for j in range(COEF):
    y = y + jnp.dot(bases[j], swt_ref[j], preferred_element_type=jnp.float32)
  y_ref[...] = y

  ssum = jnp.sum(y, axis=0)
  ssq = jnp.sum(y * y, axis=0)
  upd = jnp.concatenate(
      [ssum[None], ssq[None], jnp.zeros((6, y.shape[1]), jnp.float32)], axis=0)

  @pl.when(pid == 0)
  def _():
    sums_ref[...] = jnp.zeros_like(sums_ref)

  sums_ref[...] += upd


def _dense1(scale, p0, p1, h, bwt, swt, block):
  nblk = N_NODES // block
  return pl.pallas_call(
      _dense1_body,
      grid=(nblk,),
      in_specs=[
          pl.BlockSpec(memory_space=pltpu.SMEM),
          pl.BlockSpec((block, D), lambda i: (i, 0)),
          pl.BlockSpec((block, D), lambda i: (i, 0)),
          pl.BlockSpec((block, D), lambda i: (i, 0)),
          pl.BlockSpec((D, HID), lambda i: (0, 0)),
          pl.BlockSpec((COEF, D, HID), lambda i: (0, 0, 0)),
      ],
      out_specs=[
          pl.BlockSpec((block, HID), lambda i: (i, 0)),
          pl.BlockSpec((8, HID), lambda i: (0, 0)),
      ],
      out_shape=[
          jax.ShapeDtypeStruct((N_NODES, HID), jnp.float32),
          jax.ShapeDtypeStruct((8, HID), jnp.float32),
      ],
  )(scale, p0, p1, h, bwt, swt)


def _bn_relu_body(y_ref, sums_ref, gamma_ref, beta_ref, o_ref):
  inv_n = 1.0 / N_NODES
  mean = sums_ref[0:1, :] * inv_n
  var = sums_ref[1:2, :] * inv_n - mean * mean
  inv = lax.rsqrt(var + 1e-5) * gamma_ref[...]
  o_ref[...] = jnp.maximum((y_ref[...] - mean) * inv + beta_ref[...], 0.0)


def _bn_relu(y, sums, gamma, beta, block):
  nblk = N_NODES // block
  return pl.pallas_call(
      _bn_relu_body,
      grid=(nblk,),
      in_specs=[
          pl.BlockSpec((block, HID), lambda i: (i, 0)),
          pl.BlockSpec((8, HID), lambda i: (0, 0)),
          pl.BlockSpec((1, HID), lambda i: (0, 0)),
          pl.BlockSpec((1, HID), lambda i: (0, 0)),
      ],
      out_specs=pl.BlockSpec((block, HID), lambda i: (i, 0)),
      out_shape=jax.ShapeDtypeStruct((N_NODES, HID), jnp.float32),
  )(y, sums, gamma, beta)


def _bn_relu_clf_body(y_ref, sums_ref, gamma_ref, beta_ref, cw_ref, cb_ref,
                      o_ref):
  inv_n = 1.0 / N_NODES
  mean = sums_ref[0:1, :] * inv_n
  var = sums_ref[1:2, :] * inv_n - mean * mean
  inv = lax.rsqrt(var + 1e-5) * gamma_ref[...]
  h = jnp.maximum((y_ref[...] - mean) * inv + beta_ref[...], 0.0)
  o_ref[...] = jnp.dot(h, cw_ref[...], preferred_element_type=jnp.float32) \
      + cb_ref[...]


def _bn_relu_clf(y, sums, gamma, beta, cw_pad, cb_pad, block):
  nblk = N_NODES // block
  return pl.pallas_call(
      _bn_relu_clf_body,
      grid=(nblk,),
      in_specs=[
          pl.BlockSpec((block, HID), lambda i: (i, 0)),
          pl.BlockSpec((8, HID), lambda i: (0, 0)),
          pl.BlockSpec((1, HID), lambda i: (0, 0)),
          pl.BlockSpec((1, HID), lambda i: (0, 0)),
          pl.BlockSpec((HID, 128), lambda i: (0, 0)),
          pl.BlockSpec((1, 128), lambda i: (0, 0)),
      ],
      out_specs=pl.BlockSpec((block, 128), lambda i: (i, 0)),
      out_shape=jax.ShapeDtypeStruct((N_NODES, 128), jnp.float32),
  )(y, sums, gamma, beta, cw_pad, cb_pad)


BLOCK = 1000


def kernel(x, edge_index, eps, kan0_base, kan0_spline, kan1_base, kan1_spline,
           bn0_gamma, bn0_beta, bn1_gamma, bn1_beta, clf_w, clf_b):
  row = edge_index[0].astype(jnp.int32)
  col = edge_index[1].astype(jnp.int32)
  pad = E_PAD - N_EDGES
  pad_rows = jnp.arange(pad, dtype=jnp.int32) % N_NODES

  def _split(v, tail):
    out = jnp.concatenate([v, tail]).reshape(NW, CNT, CHUNK)
    return jnp.pad(out, ((0, 0), (0, GRP), (0, 0)))

  row_r = _split(row, pad_rows)
  col_r = _split(col, N_NODES + jnp.arange(pad, dtype=jnp.int32))
  zrows = jnp.zeros((H_AUG - N_NODES, D), jnp.float32)

  bwts = [kan0_base.T, kan1_base.T]
  swts = [kan0_spline.transpose(2, 1, 0), kan1_spline.transpose(2, 1, 0)]
  gammas = [bn0_gamma.reshape(1, HID), bn1_gamma.reshape(1, HID)]
  betas = [bn0_beta.reshape(1, HID), bn1_beta.reshape(1, HID)]
  cw_pad = jnp.zeros((HID, 128), jnp.float32).at[:, :OUT].set(clf_w.T)
  cb_pad = jnp.zeros((1, 128), jnp.float32).at[0, :OUT].set(clf_b)

  h = x
  for layer in range(2):
    h_aug = jnp.concatenate([h, zrows], axis=0)
    partials = _sc_scatter(h_aug, row_r, col_r)[:, :N_NODES]
    scale = (1.0 + eps[layer]).reshape(1)
    y, sums = _dense1(scale, partials[0], partials[1], h,
                      bwts[layer], swts[layer], BLOCK)
    if layer == 0:
      h = _bn_relu(y, sums, gammas[layer], betas[layer], BLOCK)
    else:
      logits_pad = _bn_relu_clf(y, sums, gammas[layer], betas[layer],
                                cw_pad, cb_pad, BLOCK)
  return logits_pad[:, :OUT]


# speedup vs baseline: 3.3845x; 1.0413x over previous
import functools

import jax
import jax.numpy as jnp
from jax import lax
from jax.experimental import pallas as pl
from jax.experimental.pallas import tpu as pltpu
from jax.experimental.pallas import tpu_sc as plsc

N_NODES = 10000
N_EDGES = 320000
D = 128
HID = 128
OUT = 10
GRID_SIZE = 5
SPLINE_ORDER = 3
COEF = GRID_SIZE + SPLINE_ORDER

NC = 2
NS = 16
NW = NC * NS
CHUNK = 128
CNT = 80
GRP = 8
E_PAD = NW * CNT * CHUNK
H_AUG = N_NODES + (E_PAD - N_EDGES)
ACC_ROWS = 10112
ROWS_PER_TILE = ACC_ROWS // NS

KNOTS = [0.4 * i - 2.2 for i in range(GRID_SIZE + 2 * SPLINE_ORDER + 1)]


NB = 2


STRIPE_PIECES = [CHUNK] * (ROWS_PER_TILE // CHUNK) + [ROWS_PER_TILE % CHUNK]


def _sc_scatter_body(h_hbm, row_hbm, col_hbm, out_hbm,
                     row_v, col_v, bufs, gsems, isems, jsems, acc_sh):
  c = lax.axis_index("c")
  s = lax.axis_index("s")
  wid = c * NS + s

  def zrow(r, carry):
    for q in range(D // 16):
      bufs[0][r, pl.ds(q * 16, 16)] = jnp.zeros((16,), jnp.float32)
    return carry

  lax.fori_loop(0, CHUNK, zrow, 0, unroll=False)
  off = s * ROWS_PER_TILE
  for n in STRIPE_PIECES:
    pltpu.sync_copy(bufs[0].at[pl.ds(0, n)], acc_sh.at[pl.ds(off, n)])
    off = off + n
  plsc.subcore_barrier()

  def idx_start(sl, g):
    pltpu.async_copy(row_hbm.at[wid, pl.ds(g * GRP, GRP)], row_v.at[sl],
                     isems[sl])
    pltpu.async_copy(col_hbm.at[wid, pl.ds(g * GRP, GRP)], col_v.at[sl],
                     jsems[sl])

  def idx_wait(sl, g):
    pltpu.make_async_copy(row_hbm.at[wid, pl.ds(g * GRP, GRP)],
                          row_v.at[sl], isems[sl]).wait()
    pltpu.make_async_copy(col_hbm.at[wid, pl.ds(g * GRP, GRP)],
                          col_v.at[sl], jsems[sl]).wait()

  idx_start(0, 0)

  def pair(p, carry):
    for sl in range(2):
      g = 2 * p + sl
      idx_wait(sl, g)
      idx_start(1 - sl, g + 1)
      descs = [None] * GRP
      for b in range(NB):
        descs[b] = pltpu.async_copy(h_hbm.at[col_v.at[sl, b]], bufs[b],
                                    gsems[b])
      for b in range(GRP):
        bsl = b % NB
        descs[b].wait()
        pltpu.sync_copy(bufs[bsl], acc_sh.at[row_v.at[sl, b]], add=True)
        if b + NB < GRP:
          descs[b + NB] = pltpu.async_copy(h_hbm.at[col_v.at[sl, b + NB]],
                                           bufs[bsl], gsems[bsl])
    return carry

  n_pairs = CNT // (2 * GRP)
  lax.fori_loop(0, n_pairs, pair, 0, unroll=False)
  idx_wait(0, 2 * n_pairs)
  plsc.subcore_barrier()

  off = s * ROWS_PER_TILE
  wdescs = [None, None]
  for q, n in enumerate(STRIPE_PIECES):
    bsl = q % NB
    if wdescs[bsl] is not None:
      wdescs[bsl].wait()
    pltpu.sync_copy(acc_sh.at[pl.ds(off, n)], bufs[bsl].at[pl.ds(0, n)])
    wdescs[bsl] = pltpu.async_copy(bufs[bsl].at[pl.ds(0, n)],
                                   out_hbm.at[c, pl.ds(off, n)], gsems[bsl])
    off = off + n
  for d in wdescs:
    if d is not None:
      d.wait()


@jax.jit
def _sc_scatter(h, row_r, col_r):
  mesh = plsc.VectorSubcoreMesh(core_axis_name="c", subcore_axis_name="s")
  fn = pl.kernel(
      _sc_scatter_body,
      out_type=jax.ShapeDtypeStruct((NC, ACC_ROWS, D), jnp.float32),
      mesh=mesh,
      scratch_types=[
          pltpu.VMEM((2, GRP, CHUNK), jnp.int32),
          pltpu.VMEM((2, GRP, CHUNK), jnp.int32),
          tuple(pltpu.VMEM((CHUNK, D), jnp.float32) for _ in range(NB)),
          tuple(pltpu.SemaphoreType.DMA for _ in range(NB)),
          tuple(pltpu.SemaphoreType.DMA for _ in range(2)),
          tuple(pltpu.SemaphoreType.DMA for _ in range(2)),
          pltpu.VMEM_SHARED((ACC_ROWS, D), jnp.float32),
      ],
  )
  return fn(h, row_r, col_r)


def _bspline_bases(x):
  g = KNOTS
  b = [jnp.where((x >= g[j]) & (x < g[j + 1]), 1.0, 0.0).astype(x.dtype)
       for j in range(len(g) - 1)]
  for k in range(1, SPLINE_ORDER + 1):
    nb = []
    for j in range(len(b) - 1):
      left = (x - g[j]) * (1.0 / (g[j + k] - g[j])) * b[j]
      right = (g[j + k + 1] - x) * (1.0 / (g[j + k + 1] - g[j + 1])) * b[j + 1]
      nb.append(left + right)
    b = nb
  return b


def _dense1_body(scale_ref, p0_ref, p1_ref, h_ref, bwt_ref, swt_ref,
                 y_ref, sums_ref):
  pid = pl.program_id(0)
  pooled = (p0_ref[0] + p1_ref[0]) + scale_ref[0] * h_ref[...]
  silu = pooled * jax.nn.sigmoid(pooled)
  y = jnp.dot(silu, bwt_ref[...], preferred_element_type=jnp.float32)
  bases = _bspline_bases(pooled)
  for j in range(COEF):
    y = y + jnp.dot(bases[j], swt_ref[j], preferred_element_type=jnp.float32)
  y_ref[...] = y

  ssum = jnp.sum(y, axis=0)
  ssq = jnp.sum(y * y, axis=0)
  upd = jnp.concatenate(
      [ssum[None], ssq[None], jnp.zeros((6, y.shape[1]), jnp.float32)], axis=0)

  @pl.when(pid == 0)
  def _():
    sums_ref[...] = jnp.zeros_like(sums_ref)

  sums_ref[...] += upd


def _dense1(scale, partials, h, bwt, swt, block):
  nblk = N_NODES // block
  return pl.pallas_call(
      _dense1_body,
      grid=(nblk,),
      in_specs=[
          pl.BlockSpec(memory_space=pltpu.SMEM),
          pl.BlockSpec((1, block, D), lambda i: (0, i, 0)),
          pl.BlockSpec((1, block, D), lambda i: (1, i, 0)),
          pl.BlockSpec((block, D), lambda i: (i, 0)),
          pl.BlockSpec((D, HID), lambda i: (0, 0)),
          pl.BlockSpec((COEF, D, HID), lambda i: (0, 0, 0)),
      ],
      out_specs=[
          pl.BlockSpec((block, HID), lambda i: (i, 0)),
          pl.BlockSpec((8, HID), lambda i: (0, 0)),
      ],
      out_shape=[
          jax.ShapeDtypeStruct((N_NODES, HID), jnp.float32),
          jax.ShapeDtypeStruct((8, HID), jnp.float32),
      ],
  )(scale, partials, partials, h, bwt, swt)


def _bn_relu_body(y_ref, sums_ref, gamma_ref, beta_ref, o_ref):
  inv_n = 1.0 / N_NODES
  mean = sums_ref[0:1, :] * inv_n
  var = sums_ref[1:2, :] * inv_n - mean * mean
  inv = lax.rsqrt(var + 1e-5) * gamma_ref[...]
  o_ref[...] = jnp.maximum((y_ref[...] - mean) * inv + beta_ref[...], 0.0)


def _bn_relu(y, sums, gamma, beta, block):
  nblk = N_NODES // block
  return pl.pallas_call(
      _bn_relu_body,
      grid=(nblk,),
      in_specs=[
          pl.BlockSpec((block, HID), lambda i: (i, 0)),
          pl.BlockSpec((8, HID), lambda i: (0, 0)),
          pl.BlockSpec((1, HID), lambda i: (0, 0)),
          pl.BlockSpec((1, HID), lambda i: (0, 0)),
      ],
      out_specs=pl.BlockSpec((block, HID), lambda i: (i, 0)),
      out_shape=jax.ShapeDtypeStruct((N_NODES, HID), jnp.float32),
  )(y, sums, gamma, beta)


def _bn_relu_clf_body(y_ref, sums_ref, gamma_ref, beta_ref, cw_ref, cb_ref,
                      o_ref):
  inv_n = 1.0 / N_NODES
  mean = sums_ref[0:1, :] * inv_n
  var = sums_ref[1:2, :] * inv_n - mean * mean
  inv = lax.rsqrt(var + 1e-5) * gamma_ref[...]
  h = jnp.maximum((y_ref[...] - mean) * inv + beta_ref[...], 0.0)
  o_ref[...] = jnp.dot(h, cw_ref[...], preferred_element_type=jnp.float32) \
      + cb_ref[...]


def _bn_relu_clf(y, sums, gamma, beta, cw_pad, cb_pad, block):
  nblk = N_NODES // block
  return pl.pallas_call(
      _bn_relu_clf_body,
      grid=(nblk,),
      in_specs=[
          pl.BlockSpec((block, HID), lambda i: (i, 0)),
          pl.BlockSpec((8, HID), lambda i: (0, 0)),
          pl.BlockSpec((1, HID), lambda i: (0, 0)),
          pl.BlockSpec((1, HID), lambda i: (0, 0)),
          pl.BlockSpec((HID, 128), lambda i: (0, 0)),
          pl.BlockSpec((1, 128), lambda i: (0, 0)),
      ],
      out_specs=pl.BlockSpec((block, 128), lambda i: (i, 0)),
      out_shape=jax.ShapeDtypeStruct((N_NODES, 128), jnp.float32),
  )(y, sums, gamma, beta, cw_pad, cb_pad)


BLOCK = 1000


def kernel(x, edge_index, eps, kan0_base, kan0_spline, kan1_base, kan1_spline,
           bn0_gamma, bn0_beta, bn1_gamma, bn1_beta, clf_w, clf_b):
  row = edge_index[0].astype(jnp.int32)
  col = edge_index[1].astype(jnp.int32)
  pad = E_PAD - N_EDGES
  pad_rows = jnp.arange(pad, dtype=jnp.int32) % N_NODES

  def _split(v, tail):
    out = jnp.concatenate([v, tail]).reshape(NW, CNT, CHUNK)
    return jnp.pad(out, ((0, 0), (0, GRP), (0, 0)))

  row_r = _split(row, pad_rows)
  col_r = _split(col, N_NODES + jnp.arange(pad, dtype=jnp.int32))
  zrows = jnp.zeros((H_AUG - N_NODES, D), jnp.float32)

  bwts = [kan0_base.T, kan1_base.T]
  swts = [kan0_spline.transpose(2, 1, 0), kan1_spline.transpose(2, 1, 0)]
  gammas = [bn0_gamma.reshape(1, HID), bn1_gamma.reshape(1, HID)]
  betas = [bn0_beta.reshape(1, HID), bn1_beta.reshape(1, HID)]
  cw_pad = jnp.zeros((HID, 128), jnp.float32).at[:, :OUT].set(clf_w.T)
  cb_pad = jnp.zeros((1, 128), jnp.float32).at[0, :OUT].set(clf_b)

  h = x
  for layer in range(2):
    h_aug = jnp.concatenate([h, zrows], axis=0)
    partials = _sc_scatter(h_aug, row_r, col_r)
    scale = (1.0 + eps[layer]).reshape(1)
    y, sums = _dense1(scale, partials, h, bwts[layer], swts[layer], BLOCK)
    if layer == 0:
      h = _bn_relu(y, sums, gammas[layer], betas[layer], BLOCK)
    else:
      logits_pad = _bn_relu_clf(y, sums, gammas[layer], betas[layer],
                                cw_pad, cb_pad, BLOCK)
  return logits_pad[:, :OUT]


# speedup vs baseline: 3.6427x; 1.0763x over previous
import functools

import jax
import jax.numpy as jnp
from jax import lax
from jax.experimental import pallas as pl
from jax.experimental.pallas import tpu as pltpu
from jax.experimental.pallas import tpu_sc as plsc

N_NODES = 10000
N_EDGES = 320000
D = 128
HID = 128
OUT = 10
GRID_SIZE = 5
SPLINE_ORDER = 3
COEF = GRID_SIZE + SPLINE_ORDER

NC = 2
NS = 16
NW = NC * NS
CHUNK = 128
CNT = 80
GRP = 8
E_PAD = NW * CNT * CHUNK
H_AUG = N_NODES + (E_PAD - N_EDGES)
ACC_ROWS = 10112
ROWS_PER_TILE = ACC_ROWS // NS

KNOTS = [0.4 * i - 2.2 for i in range(GRID_SIZE + 2 * SPLINE_ORDER + 1)]


NB = 2


STRIPE_PIECES = [CHUNK] * (ROWS_PER_TILE // CHUNK) + [ROWS_PER_TILE % CHUNK]


def _sc_scatter_body(h_hbm, row_hbm, col_hbm, out_hbm,
                     row_v, col_v, bufs, gsems, isems, jsems, acc_sh):
  c = lax.axis_index("c")
  s = lax.axis_index("s")
  wid = c * NS + s

  def zrow(r, carry):
    for q in range(D // 16):
      bufs[0][r, pl.ds(q * 16, 16)] = jnp.zeros((16,), jnp.float32)
    return carry

  lax.fori_loop(0, CHUNK, zrow, 0, unroll=False)
  off = s * ROWS_PER_TILE
  for n in STRIPE_PIECES:
    pltpu.sync_copy(bufs[0].at[pl.ds(0, n)], acc_sh.at[pl.ds(off, n)])
    off = off + n
  plsc.subcore_barrier()

  def idx_start(sl, g):
    pltpu.async_copy(row_hbm.at[wid, pl.ds(g * GRP, GRP)], row_v.at[sl],
                     isems[sl])
    pltpu.async_copy(col_hbm.at[wid, pl.ds(g * GRP, GRP)], col_v.at[sl],
                     jsems[sl])

  def idx_wait(sl, g):
    pltpu.make_async_copy(row_hbm.at[wid, pl.ds(g * GRP, GRP)],
                          row_v.at[sl], isems[sl]).wait()
    pltpu.make_async_copy(col_hbm.at[wid, pl.ds(g * GRP, GRP)],
                          col_v.at[sl], jsems[sl]).wait()

  idx_start(0, 0)

  def pair(p, carry):
    for sl in range(2):
      g = 2 * p + sl
      idx_wait(sl, g)
      idx_start(1 - sl, g + 1)
      descs = [None] * GRP
      for b in range(NB):
        descs[b] = pltpu.async_copy(h_hbm.at[col_v.at[sl, b]], bufs[b],
                                    gsems[b])
      for b in range(GRP):
        bsl = b % NB
        descs[b].wait()
        pltpu.sync_copy(bufs[bsl], acc_sh.at[row_v.at[sl, b]], add=True)
        if b + NB < GRP:
          descs[b + NB] = pltpu.async_copy(h_hbm.at[col_v.at[sl, b + NB]],
                                           bufs[bsl], gsems[bsl])
    return carry

  n_pairs = CNT // (2 * GRP)
  lax.fori_loop(0, n_pairs, pair, 0, unroll=False)
  idx_wait(0, 2 * n_pairs)
  plsc.subcore_barrier()

  off = s * ROWS_PER_TILE
  wdescs = [None, None]
  for q, n in enumerate(STRIPE_PIECES):
    bsl = q % NB
    if wdescs[bsl] is not None:
      wdescs[bsl].wait()
    pltpu.sync_copy(acc_sh.at[pl.ds(off, n)], bufs[bsl].at[pl.ds(0, n)])
    wdescs[bsl] = pltpu.async_copy(bufs[bsl].at[pl.ds(0, n)],
                                   out_hbm.at[c, pl.ds(off, n)], gsems[bsl])
    off = off + n
  for d in wdescs:
    if d is not None:
      d.wait()


@jax.jit
def _sc_scatter(h, row_r, col_r):
  mesh = plsc.VectorSubcoreMesh(core_axis_name="c", subcore_axis_name="s")
  fn = pl.kernel(
      _sc_scatter_body,
      out_type=jax.ShapeDtypeStruct((NC, ACC_ROWS, D), jnp.float32),
      mesh=mesh,
      scratch_types=[
          pltpu.VMEM((2, GRP, CHUNK), jnp.int32),
          pltpu.VMEM((2, GRP, CHUNK), jnp.int32),
          tuple(pltpu.VMEM((CHUNK, D), jnp.float32) for _ in range(NB)),
          tuple(pltpu.SemaphoreType.DMA for _ in range(NB)),
          tuple(pltpu.SemaphoreType.DMA for _ in range(2)),
          tuple(pltpu.SemaphoreType.DMA for _ in range(2)),
          pltpu.VMEM_SHARED((ACC_ROWS, D), jnp.float32),
      ],
  )
  return fn(h, row_r, col_r)


def _bspline_bases(x):
  t = x * 2.5 + 5.5
  m = jnp.floor(t)
  f = t - m
  f2 = f * f
  f3 = f2 * f
  w3 = f3 * (1.0 / 6.0)
  w0 = (1.0 / 6.0) - 0.5 * f + 0.5 * f2 - w3
  w2 = (1.0 / 6.0) + 0.5 * f + 0.5 * f2 - 0.5 * f3
  w1 = (2.0 / 3.0) - f2 + 0.5 * f3
  msk = [m == float(cc) for cc in range(11)]
  zero = jnp.zeros_like(x)
  ws = (w3, w2, w1, w0)

  def sel(c, w):
    return jnp.where(msk[c], w, zero) if 0 <= c <= 10 else zero

  out = []
  for j in range(COEF):
    b = sel(j, w3)
    for k in range(1, 4):
      b = b + sel(j + k, ws[k])
    out.append(b)
  return out


def _dense1_body(scale_ref, p0_ref, p1_ref, h_ref, bwt_ref, swt_ref,
                 y_ref, sums_ref):
  pid = pl.program_id(0)
  pooled = (p0_ref[0] + p1_ref[0]) + scale_ref[0] * h_ref[...]
  silu = pooled * jax.nn.sigmoid(pooled)
  y = jnp.dot(silu, bwt_ref[...], preferred_element_type=jnp.float32)
  bases = _bspline_bases(pooled)
  for j in range(COEF):
    y = y + jnp.dot(bases[j], swt_ref[j], preferred_element_type=jnp.float32)
  y_ref[...] = y

  ssum = jnp.sum(y, axis=0)
  ssq = jnp.sum(y * y, axis=0)
  upd = jnp.concatenate(
      [ssum[None], ssq[None], jnp.zeros((6, y.shape[1]), jnp.float32)], axis=0)

  @pl.when(pid == 0)
  def _():
    sums_ref[...] = jnp.zeros_like(sums_ref)

  sums_ref[...] += upd


def _dense1(scale, partials, h, bwt, swt, block):
  nblk = N_NODES // block
  return pl.pallas_call(
      _dense1_body,
      grid=(nblk,),
      in_specs=[
          pl.BlockSpec(memory_space=pltpu.SMEM),
          pl.BlockSpec((1, block, D), lambda i: (0, i, 0)),
          pl.BlockSpec((1, block, D), lambda i: (1, i, 0)),
          pl.BlockSpec((block, D), lambda i: (i, 0)),
          pl.BlockSpec((D, HID), lambda i: (0, 0)),
          pl.BlockSpec((COEF, D, HID), lambda i: (0, 0, 0)),
      ],
      out_specs=[
          pl.BlockSpec((block, HID), lambda i: (i, 0)),
          pl.BlockSpec((8, HID), lambda i: (0, 0)),
      ],
      out_shape=[
          jax.ShapeDtypeStruct((N_NODES, HID), jnp.float32),
          jax.ShapeDtypeStruct((8, HID), jnp.float32),
      ],
  )(scale, partials, partials, h, bwt, swt)


def _bn_relu_body(y_ref, sums_ref, gamma_ref, beta_ref, o_ref):
  inv_n = 1.0 / N_NODES
  mean = sums_ref[0:1, :] * inv_n
  var = sums_ref[1:2, :] * inv_n - mean * mean
  inv = lax.rsqrt(var + 1e-5) * gamma_ref[...]
  o_ref[...] = jnp.maximum((y_ref[...] - mean) * inv + beta_ref[...], 0.0)


def _bn_relu(y, sums, gamma, beta, block):
  nblk = N_NODES // block
  return pl.pallas_call(
      _bn_relu_body,
      grid=(nblk,),
      in_specs=[
          pl.BlockSpec((block, HID), lambda i: (i, 0)),
          pl.BlockSpec((8, HID), lambda i: (0, 0)),
          pl.BlockSpec((1, HID), lambda i: (0, 0)),
          pl.BlockSpec((1, HID), lambda i: (0, 0)),
      ],
      out_specs=pl.BlockSpec((block, HID), lambda i: (i, 0)),
      out_shape=jax.ShapeDtypeStruct((N_NODES, HID), jnp.float32),
  )(y, sums, gamma, beta)


def _bn_relu_clf_body(y_ref, sums_ref, gamma_ref, beta_ref, cw_ref, cb_ref,
                      o_ref):
  inv_n = 1.0 / N_NODES
  mean = sums_ref[0:1, :] * inv_n
  var = sums_ref[1:2, :] * inv_n - mean * mean
  inv = lax.rsqrt(var + 1e-5) * gamma_ref[...]
  h = jnp.maximum((y_ref[...] - mean) * inv + beta_ref[...], 0.0)
  o_ref[...] = jnp.dot(h, cw_ref[...], preferred_element_type=jnp.float32) \
      + cb_ref[...]


def _bn_relu_clf(y, sums, gamma, beta, cw_pad, cb_pad, block):
  nblk = N_NODES // block
  return pl.pallas_call(
      _bn_relu_clf_body,
      grid=(nblk,),
      in_specs=[
          pl.BlockSpec((block, HID), lambda i: (i, 0)),
          pl.BlockSpec((8, HID), lambda i: (0, 0)),
          pl.BlockSpec((1, HID), lambda i: (0, 0)),
          pl.BlockSpec((1, HID), lambda i: (0, 0)),
          pl.BlockSpec((HID, 128), lambda i: (0, 0)),
          pl.BlockSpec((1, 128), lambda i: (0, 0)),
      ],
      out_specs=pl.BlockSpec((block, 128), lambda i: (i, 0)),
      out_shape=jax.ShapeDtypeStruct((N_NODES, 128), jnp.float32),
  )(y, sums, gamma, beta, cw_pad, cb_pad)


BLOCK = 1000


def kernel(x, edge_index, eps, kan0_base, kan0_spline, kan1_base, kan1_spline,
           bn0_gamma, bn0_beta, bn1_gamma, bn1_beta, clf_w, clf_b):
  row = edge_index[0].astype(jnp.int32)
  col = edge_index[1].astype(jnp.int32)
  pad = E_PAD - N_EDGES
  pad_rows = jnp.arange(pad, dtype=jnp.int32) % N_NODES

  def _split(v, tail):
    out = jnp.concatenate([v, tail]).reshape(NW, CNT, CHUNK)
    return jnp.pad(out, ((0, 0), (0, GRP), (0, 0)))

  row_r = _split(row, pad_rows)
  col_r = _split(col, N_NODES + jnp.arange(pad, dtype=jnp.int32))
  zrows = jnp.zeros((H_AUG - N_NODES, D), jnp.float32)

  bwts = [kan0_base.T, kan1_base.T]
  swts = [kan0_spline.transpose(2, 1, 0), kan1_spline.transpose(2, 1, 0)]
  gammas = [bn0_gamma.reshape(1, HID), bn1_gamma.reshape(1, HID)]
  betas = [bn0_beta.reshape(1, HID), bn1_beta.reshape(1, HID)]
  cw_pad = jnp.zeros((HID, 128), jnp.float32).at[:, :OUT].set(clf_w.T)
  cb_pad = jnp.zeros((1, 128), jnp.float32).at[0, :OUT].set(clf_b)

  h = x
  for layer in range(2):
    h_aug = jnp.concatenate([h, zrows], axis=0)
    partials = _sc_scatter(h_aug, row_r, col_r)
    scale = (1.0 + eps[layer]).reshape(1)
    y, sums = _dense1(scale, partials, h, bwts[layer], swts[layer], BLOCK)
    if layer == 0:
      h = _bn_relu(y, sums, gammas[layer], betas[layer], BLOCK)
    else:
      logits_pad = _bn_relu_clf(y, sums, gammas[layer], betas[layer],
                                cw_pad, cb_pad, BLOCK)
  return logits_pad[:, :OUT]


# speedup vs baseline: 3.7048x; 1.0171x over previous
import functools

import jax
import jax.numpy as jnp
from jax import lax
from jax.experimental import pallas as pl
from jax.experimental.pallas import tpu as pltpu
from jax.experimental.pallas import tpu_sc as plsc

N_NODES = 10000
N_EDGES = 320000
D = 128
HID = 128
OUT = 10
GRID_SIZE = 5
SPLINE_ORDER = 3
COEF = GRID_SIZE + SPLINE_ORDER

NC = 2
NS = 16
NW = NC * NS
CHUNK = 128
CNT = 80
GRP = 8
E_PAD = NW * CNT * CHUNK
H_AUG = N_NODES + (E_PAD - N_EDGES)
ACC_ROWS = 10112
ROWS_PER_TILE = ACC_ROWS // NS

KNOTS = [0.4 * i - 2.2 for i in range(GRID_SIZE + 2 * SPLINE_ORDER + 1)]


NB = 2


STRIPE_PIECES = [CHUNK] * (ROWS_PER_TILE // CHUNK) + [ROWS_PER_TILE % CHUNK]


def _sc_scatter_body(h_hbm, row_hbm, col_hbm, out_hbm,
                     row_v, col_v, bufs, gsems, isems, jsems, acc_sh):
  c = lax.axis_index("c")
  s = lax.axis_index("s")
  wid = c * NS + s

  def zrow(r, carry):
    for q in range(D // 16):
      bufs[0][r, pl.ds(q * 16, 16)] = jnp.zeros((16,), jnp.float32)
    return carry

  lax.fori_loop(0, CHUNK, zrow, 0, unroll=False)
  off = s * ROWS_PER_TILE
  for n in STRIPE_PIECES:
    pltpu.sync_copy(bufs[0].at[pl.ds(0, n)], acc_sh.at[pl.ds(off, n)])
    off = off + n
  plsc.subcore_barrier()

  def idx_start(sl, g):
    pltpu.async_copy(row_hbm.at[wid, pl.ds(g * GRP, GRP)], row_v.at[sl],
                     isems[sl])
    pltpu.async_copy(col_hbm.at[wid, pl.ds(g * GRP, GRP)], col_v.at[sl],
                     jsems[sl])

  def idx_wait(sl, g):
    pltpu.make_async_copy(row_hbm.at[wid, pl.ds(g * GRP, GRP)],
                          row_v.at[sl], isems[sl]).wait()
    pltpu.make_async_copy(col_hbm.at[wid, pl.ds(g * GRP, GRP)],
                          col_v.at[sl], jsems[sl]).wait()

  idx_start(0, 0)

  def pair(p, carry):
    for sl in range(2):
      g = 2 * p + sl
      idx_wait(sl, g)
      idx_start(1 - sl, g + 1)
      descs = [None] * GRP
      for b in range(NB):
        descs[b] = pltpu.async_copy(h_hbm.at[col_v.at[sl, b]], bufs[b],
                                    gsems[b])
      for b in range(GRP):
        bsl = b % NB
        descs[b].wait()
        pltpu.sync_copy(bufs[bsl], acc_sh.at[row_v.at[sl, b]], add=True)
        if b + NB < GRP:
          descs[b + NB] = pltpu.async_copy(h_hbm.at[col_v.at[sl, b + NB]],
                                           bufs[bsl], gsems[bsl])
    return carry

  n_pairs = CNT // (2 * GRP)
  lax.fori_loop(0, n_pairs, pair, 0, unroll=False)
  idx_wait(0, 2 * n_pairs)
  plsc.subcore_barrier()

  off = s * ROWS_PER_TILE
  wdescs = [None, None]
  for q, n in enumerate(STRIPE_PIECES):
    bsl = q % NB
    if wdescs[bsl] is not None:
      wdescs[bsl].wait()
    pltpu.sync_copy(acc_sh.at[pl.ds(off, n)], bufs[bsl].at[pl.ds(0, n)])
    wdescs[bsl] = pltpu.async_copy(bufs[bsl].at[pl.ds(0, n)],
                                   out_hbm.at[c, pl.ds(off, n)], gsems[bsl])
    off = off + n
  for d in wdescs:
    if d is not None:
      d.wait()


@jax.jit
def _sc_scatter(h, row_r, col_r):
  mesh = plsc.VectorSubcoreMesh(core_axis_name="c", subcore_axis_name="s")
  fn = pl.kernel(
      _sc_scatter_body,
      out_type=jax.ShapeDtypeStruct((NC, ACC_ROWS, D), jnp.float32),
      mesh=mesh,
      scratch_types=[
          pltpu.VMEM((2, GRP, CHUNK), jnp.int32),
          pltpu.VMEM((2, GRP, CHUNK), jnp.int32),
          tuple(pltpu.VMEM((CHUNK, D), jnp.float32) for _ in range(NB)),
          tuple(pltpu.SemaphoreType.DMA for _ in range(NB)),
          tuple(pltpu.SemaphoreType.DMA for _ in range(2)),
          tuple(pltpu.SemaphoreType.DMA for _ in range(2)),
          pltpu.VMEM_SHARED((ACC_ROWS, D), jnp.float32),
      ],
  )
  return fn(h, row_r, col_r)


def _bspline_bases(x):
  t = x * 2.5 + 5.5
  m = jnp.floor(t)
  f = t - m
  f2 = f * f
  f3 = f2 * f
  w3 = f3 * (1.0 / 6.0)
  w0 = (1.0 / 6.0) - 0.5 * f + 0.5 * f2 - w3
  w2 = (1.0 / 6.0) + 0.5 * f + 0.5 * f2 - 0.5 * f3
  w1 = (2.0 / 3.0) - f2 + 0.5 * f3
  msk = [m == float(cc) for cc in range(11)]
  zero = jnp.zeros_like(x)
  ws = (w3, w2, w1, w0)

  def sel(c, w):
    return jnp.where(msk[c], w, zero) if 0 <= c <= 10 else zero

  out = []
  for j in range(COEF):
    b = sel(j, w3)
    for k in range(1, 4):
      b = b + sel(j + k, ws[k])
    out.append(b)
  return out


def _dense1_body(scale_ref, p0_ref, p1_ref, h_ref, bwt_ref, swt_ref,
                 y_ref, sums_ref):
  pid = pl.program_id(0)
  pooled = (p0_ref[0] + p1_ref[0]) + scale_ref[0] * h_ref[...]
  silu = pooled * jax.nn.sigmoid(pooled)
  y = jnp.dot(silu, bwt_ref[...], preferred_element_type=jnp.float32)
  bases = _bspline_bases(pooled)
  for j in range(COEF):
    y = y + jnp.dot(bases[j], swt_ref[j], preferred_element_type=jnp.float32)
  y_ref[...] = y

  ssum = jnp.sum(y, axis=0)
  ssq = jnp.sum(y * y, axis=0)
  upd = jnp.concatenate(
      [ssum[None], ssq[None], jnp.zeros((6, y.shape[1]), jnp.float32)], axis=0)

  @pl.when(pid == 0)
  def _():
    sums_ref[...] = jnp.zeros_like(sums_ref)

  sums_ref[...] += upd


def _dense1(scale, partials, h, bwt, swt, block):
  nblk = N_NODES // block
  return pl.pallas_call(
      _dense1_body,
      grid=(nblk,),
      in_specs=[
          pl.BlockSpec(memory_space=pltpu.SMEM),
          pl.BlockSpec((1, block, D), lambda i: (0, i, 0)),
          pl.BlockSpec((1, block, D), lambda i: (1, i, 0)),
          pl.BlockSpec((block, D), lambda i: (i, 0)),
          pl.BlockSpec((D, HID), lambda i: (0, 0)),
          pl.BlockSpec((COEF, D, HID), lambda i: (0, 0, 0)),
      ],
      out_specs=[
          pl.BlockSpec((block, HID), lambda i: (i, 0)),
          pl.BlockSpec((8, HID), lambda i: (0, 0)),
      ],
      out_shape=[
          jax.ShapeDtypeStruct((N_NODES, HID), jnp.float32),
          jax.ShapeDtypeStruct((8, HID), jnp.float32),
      ],
  )(scale, partials, partials, h, bwt, swt)


def _bn_relu_body(y_ref, sums_ref, gamma_ref, beta_ref, o_ref):
  inv_n = 1.0 / N_NODES
  mean = sums_ref[0:1, :] * inv_n
  var = sums_ref[1:2, :] * inv_n - mean * mean
  inv = lax.rsqrt(var + 1e-5) * gamma_ref[...]
  o_ref[...] = jnp.maximum((y_ref[...] - mean) * inv + beta_ref[...], 0.0)


def _bn_relu(y, sums, gamma, beta, block):
  nblk = N_NODES // block
  return pl.pallas_call(
      _bn_relu_body,
      grid=(nblk,),
      in_specs=[
          pl.BlockSpec((block, HID), lambda i: (i, 0)),
          pl.BlockSpec((8, HID), lambda i: (0, 0)),
          pl.BlockSpec((1, HID), lambda i: (0, 0)),
          pl.BlockSpec((1, HID), lambda i: (0, 0)),
      ],
      out_specs=pl.BlockSpec((block, HID), lambda i: (i, 0)),
      out_shape=jax.ShapeDtypeStruct((N_NODES, HID), jnp.float32),
  )(y, sums, gamma, beta)


def _bn_relu_clf_body(y_ref, sums_ref, gamma_ref, beta_ref, cw_ref, cb_ref,
                      o_ref):
  inv_n = 1.0 / N_NODES
  mean = sums_ref[0:1, :] * inv_n
  var = sums_ref[1:2, :] * inv_n - mean * mean
  inv = lax.rsqrt(var + 1e-5) * gamma_ref[...]
  h = jnp.maximum((y_ref[...] - mean) * inv + beta_ref[...], 0.0)
  o_ref[...] = jnp.dot(h, cw_ref[...], preferred_element_type=jnp.float32) \
      + cb_ref[...]


def _bn_relu_clf(y, sums, gamma, beta, cw_pad, cb_pad, block):
  nblk = N_NODES // block
  return pl.pallas_call(
      _bn_relu_clf_body,
      grid=(nblk,),
      in_specs=[
          pl.BlockSpec((block, HID), lambda i: (i, 0)),
          pl.BlockSpec((8, HID), lambda i: (0, 0)),
          pl.BlockSpec((1, HID), lambda i: (0, 0)),
          pl.BlockSpec((1, HID), lambda i: (0, 0)),
          pl.BlockSpec((HID, 128), lambda i: (0, 0)),
          pl.BlockSpec((1, 128), lambda i: (0, 0)),
      ],
      out_specs=pl.BlockSpec((block, 128), lambda i: (i, 0)),
      out_shape=jax.ShapeDtypeStruct((N_NODES, 128), jnp.float32),
  )(y, sums, gamma, beta, cw_pad, cb_pad)


BLOCK = 2000


def kernel(x, edge_index, eps, kan0_base, kan0_spline, kan1_base, kan1_spline,
           bn0_gamma, bn0_beta, bn1_gamma, bn1_beta, clf_w, clf_b):
  row = edge_index[0].astype(jnp.int32)
  col = edge_index[1].astype(jnp.int32)
  pad = E_PAD - N_EDGES
  pad_rows = jnp.arange(pad, dtype=jnp.int32) % N_NODES

  def _split(v, tail):
    out = jnp.concatenate([v, tail]).reshape(NW, CNT, CHUNK)
    return jnp.pad(out, ((0, 0), (0, GRP), (0, 0)))

  row_r = _split(row, pad_rows)
  col_r = _split(col, N_NODES + jnp.arange(pad, dtype=jnp.int32))
  zrows = jnp.zeros((H_AUG - N_NODES, D), jnp.float32)

  bwts = [kan0_base.T, kan1_base.T]
  swts = [kan0_spline.transpose(2, 1, 0), kan1_spline.transpose(2, 1, 0)]
  gammas = [bn0_gamma.reshape(1, HID), bn1_gamma.reshape(1, HID)]
  betas = [bn0_beta.reshape(1, HID), bn1_beta.reshape(1, HID)]
  cw_pad = jnp.zeros((HID, 128), jnp.float32).at[:, :OUT].set(clf_w.T)
  cb_pad = jnp.zeros((1, 128), jnp.float32).at[0, :OUT].set(clf_b)

  h = x
  for layer in range(2):
    h_aug = jnp.concatenate([h, zrows], axis=0)
    partials = _sc_scatter(h_aug, row_r, col_r)
    scale = (1.0 + eps[layer]).reshape(1)
    y, sums = _dense1(scale, partials, h, bwts[layer], swts[layer], BLOCK)
    if layer == 0:
      h = _bn_relu(y, sums, gammas[layer], betas[layer], BLOCK)
    else:
      logits_pad = _bn_relu_clf(y, sums, gammas[layer], betas[layer],
                                cw_pad, cb_pad, BLOCK)
  return logits_pad[:, :OUT]
